# baseline (speedup 1.0000x reference)
import functools

import jax
import jax.numpy as jnp
from jax import lax
from jax.experimental import pallas as pl
from jax.experimental.pallas import tpu as pltpu
from jax.experimental.pallas import tpu_sc as plsc

_M0 = 0.3
_K = 308
_N_TILE = 128


def _dtm_body(x_ref, g_ref, o_ref, *, k, weight_bound, n_iters):
    x = x_ref[0]
    x0 = x[:, 0:1]
    x1 = x[:, 1:2]
    g0 = g_ref[0:1, :]
    g1 = g_ref[1:2, :]
    dx = x0 - g0
    dy = x1 - g1
    d2 = dx * dx + dy * dy
    d2i = jax.lax.bitcast_convert_type(d2, jnp.int32)

    lo0 = jnp.zeros(g0.shape, jnp.int32)
    hi0 = jnp.full(g0.shape, 0x7F800000, jnp.int32)

    def step(_, carry):
        lo, hi = carry
        mid = lo + ((hi - lo) >> 1)
        cnt = jnp.sum((d2i <= mid).astype(jnp.int32), axis=0, keepdims=True)
        ge = cnt >= k
        return jnp.where(ge, lo, mid + 1), jnp.where(ge, mid, hi)

    lo, _ = jax.lax.fori_loop(0, n_iters, step, (lo0, hi0))
    t = jax.lax.bitcast_convert_type(lo, jnp.float32)

    less = d2 < t
    cnt_less = jnp.sum(less.astype(jnp.float32), axis=0, keepdims=True)
    sum_less = jnp.sum(jnp.where(less, d2, 0.0), axis=0, keepdims=True)
    dtm = jnp.sqrt((sum_less + (weight_bound - cnt_less) * t) / weight_bound)
    o_ref[0] = dtm


def _tc_dtm(inputs, grid_pts):
    B, M, d = inputs.shape
    N = grid_pts.shape[0]
    weight_bound = _M0 * M
    n_pad = pl.cdiv(N, _N_TILE) * _N_TILE

    gT = jnp.zeros((8, n_pad), jnp.float32)
    gT = gT.at[0, :N].set(grid_pts[:, 0]).at[1, :N].set(grid_pts[:, 1])

    body = functools.partial(
        _dtm_body, k=_K, weight_bound=weight_bound, n_iters=31)
    out = pl.pallas_call(
        body,
        grid=(B, n_pad // _N_TILE),
        in_specs=[
            pl.BlockSpec((1, M, d), lambda b, j: (b, 0, 0)),
            pl.BlockSpec((8, _N_TILE), lambda b, j: (0, j)),
        ],
        out_specs=pl.BlockSpec((1, 1, _N_TILE), lambda b, j: (b, 0, j)),
        out_shape=jax.ShapeDtypeStruct((B, 1, n_pad), jnp.float32),
    )(inputs, gT)
    return out[:, 0, :N]


_L = 16
_NW = 32
_UD = 8
_US = 16


def _sc_dtm(xs, ys, gx, gy, *, n_chunks_pb, chunks_per_w):
    B, M = xs.shape
    NP = gx.shape[0]
    k = _K
    wb = _M0 * M
    total_chunks = B * n_chunks_pb
    mesh = plsc.VectorSubcoreMesh(core_axis_name="c", subcore_axis_name="s")

    @functools.partial(
        pl.kernel,
        mesh=mesh,
        out_type=jax.ShapeDtypeStruct((B, NP), jnp.float32),
        scratch_types=[
            pltpu.VMEM((M,), jnp.float32),
            pltpu.VMEM((M,), jnp.float32),
            pltpu.VMEM((NP,), jnp.float32),
            pltpu.VMEM((NP,), jnp.float32),
            pltpu.VMEM((M * _L,), jnp.float32),
            pltpu.VMEM((_L,), jnp.float32),
        ],
    )
    def sc_kernel(xs_h, ys_h, gx_h, gy_h, out_h, x_v, y_v, gx_v, gy_v, d_v, o_v):
        wid = lax.axis_index("s") * 2 + lax.axis_index("c")
        pltpu.sync_copy(gx_h, gx_v)
        pltpu.sync_copy(gy_h, gy_v)

        def chunk_body(i, _):
            cid = i * _NW + wid
            b = cid // n_chunks_pb
            cb = cid - b * n_chunks_pb
            pltpu.sync_copy(xs_h.at[b], x_v)
            pltpu.sync_copy(ys_h.at[b], y_v)
            gxc = gx_v[pl.ds(cb * _L, _L)]
            gyc = gy_v[pl.ds(cb * _L, _L)]

            def dist_body(jj, _):
                base = jj * _L
                xc = x_v[pl.ds(base, _L)]
                yc = y_v[pl.ds(base, _L)]
                for u in range(_L):
                    idx = jnp.full((_L,), u, jnp.int32)
                    xj = xc.at[idx].get(mode="promise_in_bounds")
                    yj = yc.at[idx].get(mode="promise_in_bounds")
                    dx = xj - gxc
                    dy = yj - gyc
                    d_v[pl.ds((base + u) * _L, _L)] = dx * dx + dy * dy
                return 0
            lax.fori_loop(0, M // _L, dist_body, 0)

            def search_step(s, carry):
                lo, hi = carry
                mid = lo + lax.shift_right_logical(hi - lo, 1)

                def cnt_body(jj, cnt):
                    for u in range(_US):
                        j = jj * _US + u
                        di = lax.bitcast_convert_type(
                            d_v[pl.ds(j * _L, _L)], jnp.int32)
                        cnt = cnt + jnp.where(di <= mid, 1, 0)
                    return cnt
                cnt = lax.fori_loop(
                    0, M // _US, cnt_body, jnp.zeros((_L,), jnp.int32))
                ge = cnt >= k
                return jnp.where(ge, lo, mid + 1), jnp.where(ge, mid, hi)

            lo, _hi = lax.fori_loop(
                0, 31, search_step,
                (jnp.zeros((_L,), jnp.int32),
                 jnp.full((_L,), 0x7F800000, jnp.int32)))
            t = lax.bitcast_convert_type(lo, jnp.float32)

            def fin_body(jj, carry):
                cl, sl = carry
                for u in range(_US):
                    j = jj * _US + u
                    dvec = d_v[pl.ds(j * _L, _L)]
                    less = dvec < t
                    cl = cl + jnp.where(less, 1.0, 0.0)
                    sl = sl + jnp.where(less, dvec, 0.0)
                return cl, sl
            cl, sl = lax.fori_loop(
                0, M // _US, fin_body,
                (jnp.zeros((_L,), jnp.float32), jnp.zeros((_L,), jnp.float32)))

            z = (sl + (wb - cl) * t) * (1.0 / wb)
            zb = lax.bitcast_convert_type(z, jnp.int32)
            y = lax.bitcast_convert_type(
                0x5F3759DF - lax.shift_right_logical(zb, 1), jnp.float32)
            for _r in range(3):
                y = y * (1.5 - 0.5 * z * y * y)
            o_v[...] = z * y
            pltpu.sync_copy(o_v, out_h.at[b, pl.ds(cb * _L, _L)])
            return 0

        lax.fori_loop(0, chunks_per_w, chunk_body, 0)

    return sc_kernel(xs, ys, gx, gy)


def _sc_dtm_full(inputs, grid_pts):
    B, M, _d = inputs.shape
    N = grid_pts.shape[0]
    n_chunks_pb = 70
    chunks_per_w = B * n_chunks_pb // _NW
    NP = n_chunks_pb * _L
    xs = inputs[:, :, 0]
    ys = inputs[:, :, 1]
    gx = jnp.zeros((NP,), jnp.float32).at[:N].set(grid_pts[:, 0])
    gy = jnp.zeros((NP,), jnp.float32).at[:N].set(grid_pts[:, 1])
    out = _sc_dtm(xs, ys, gx, gy,
                  n_chunks_pb=n_chunks_pb, chunks_per_w=chunks_per_w)
    return out[:, :N]


def kernel(inputs, grid):
    return _sc_dtm_full(inputs, grid)


# speedup vs baseline: 1.8727x; 1.8727x over previous
import functools

import jax
import jax.numpy as jnp
from jax import lax
from jax.experimental import pallas as pl
from jax.experimental.pallas import tpu as pltpu
from jax.experimental.pallas import tpu_sc as plsc

_M0 = 0.3
_K = 308
_N_TILE = 128


def _dtm_body(x_ref, g_ref, o_ref, *, k, weight_bound, n_iters):
    x = x_ref[0]
    x0 = x[:, 0:1]
    x1 = x[:, 1:2]
    g0 = g_ref[0:1, :]
    g1 = g_ref[1:2, :]
    dx = x0 - g0
    dy = x1 - g1
    d2 = dx * dx + dy * dy
    d2i = jax.lax.bitcast_convert_type(d2, jnp.int32)

    lo0 = jnp.zeros(g0.shape, jnp.int32)
    hi0 = jnp.full(g0.shape, 0x7F800000, jnp.int32)

    def step(_, carry):
        lo, hi = carry
        mid = lo + ((hi - lo) >> 1)
        cnt = jnp.sum((d2i <= mid).astype(jnp.int32), axis=0, keepdims=True)
        ge = cnt >= k
        return jnp.where(ge, lo, mid + 1), jnp.where(ge, mid, hi)

    lo, _ = jax.lax.fori_loop(0, n_iters, step, (lo0, hi0))
    t = jax.lax.bitcast_convert_type(lo, jnp.float32)

    less = d2 < t
    cnt_less = jnp.sum(less.astype(jnp.float32), axis=0, keepdims=True)
    sum_less = jnp.sum(jnp.where(less, d2, 0.0), axis=0, keepdims=True)
    dtm = jnp.sqrt((sum_less + (weight_bound - cnt_less) * t) / weight_bound)
    o_ref[0] = dtm


def _tc_dtm(inputs, grid_pts):
    B, M, d = inputs.shape
    N = grid_pts.shape[0]
    weight_bound = _M0 * M
    n_pad = pl.cdiv(N, _N_TILE) * _N_TILE

    gT = jnp.zeros((8, n_pad), jnp.float32)
    gT = gT.at[0, :N].set(grid_pts[:, 0]).at[1, :N].set(grid_pts[:, 1])

    body = functools.partial(
        _dtm_body, k=_K, weight_bound=weight_bound, n_iters=31)
    out = pl.pallas_call(
        body,
        grid=(B, n_pad // _N_TILE),
        in_specs=[
            pl.BlockSpec((1, M, d), lambda b, j: (b, 0, 0)),
            pl.BlockSpec((8, _N_TILE), lambda b, j: (0, j)),
        ],
        out_specs=pl.BlockSpec((1, 1, _N_TILE), lambda b, j: (b, 0, j)),
        out_shape=jax.ShapeDtypeStruct((B, 1, n_pad), jnp.float32),
    )(inputs, gT)
    return out[:, 0, :N]


_L = 16
_NW = 32
_UD = 8
_US = 16


def _sc_dtm(xs, ys, gx, gy, *, n_chunks_pb, chunks_per_w):
    B, M = xs.shape
    NP = gx.shape[0]
    k = _K
    wb = _M0 * M
    total_chunks = B * n_chunks_pb
    mesh = plsc.VectorSubcoreMesh(core_axis_name="c", subcore_axis_name="s")

    @functools.partial(
        pl.kernel,
        mesh=mesh,
        out_type=jax.ShapeDtypeStruct((B, NP), jnp.float32),
        scratch_types=[
            pltpu.VMEM((M,), jnp.float32),
            pltpu.VMEM((M,), jnp.float32),
            pltpu.VMEM((NP,), jnp.float32),
            pltpu.VMEM((NP,), jnp.float32),
            pltpu.VMEM((M * _L,), jnp.float32),
            pltpu.VMEM((_L,), jnp.float32),
        ],
    )
    def sc_kernel(xs_h, ys_h, gx_h, gy_h, out_h, x_v, y_v, gx_v, gy_v, d_v, o_v):
        wid = lax.axis_index("s") * 2 + lax.axis_index("c")
        pltpu.sync_copy(gx_h, gx_v)
        pltpu.sync_copy(gy_h, gy_v)

        def chunk_body(i, _):
            cid = i * _NW + wid
            b = cid // n_chunks_pb
            cb = cid - b * n_chunks_pb
            pltpu.sync_copy(xs_h.at[b], x_v)
            pltpu.sync_copy(ys_h.at[b], y_v)
            gxc = gx_v[pl.ds(cb * _L, _L)]
            gyc = gy_v[pl.ds(cb * _L, _L)]

            def dist_body(jj, _):
                base = jj * _L
                xc = x_v[pl.ds(base, _L)]
                yc = y_v[pl.ds(base, _L)]
                for u in range(_L):
                    idx = jnp.full((_L,), u, jnp.int32)
                    xj = xc.at[idx].get(mode="promise_in_bounds")
                    yj = yc.at[idx].get(mode="promise_in_bounds")
                    dx = xj - gxc
                    dy = yj - gyc
                    d_v[pl.ds((base + u) * _L, _L)] = dx * dx + dy * dy
                return 0
            lax.fori_loop(0, M // _L, dist_body, 0)

            def search_step(s, carry):
                lo, hi = carry
                mid = lo + lax.shift_right_logical(hi - lo, 1)

                def cnt_body(jj, cnt):
                    for u in range(_US):
                        j = jj * _US + u
                        di = lax.bitcast_convert_type(
                            d_v[pl.ds(j * _L, _L)], jnp.int32)
                        cnt = cnt + jnp.where(di <= mid, 1, 0)
                    return cnt
                cnt = lax.fori_loop(
                    0, M // _US, cnt_body, jnp.zeros((_L,), jnp.int32))
                ge = cnt >= k
                return jnp.where(ge, lo, mid + 1), jnp.where(ge, mid, hi)

            lo, _hi = lax.fori_loop(
                0, 31, search_step,
                (jnp.zeros((_L,), jnp.int32),
                 jnp.full((_L,), 0x7F800000, jnp.int32)))
            t = lax.bitcast_convert_type(lo, jnp.float32)

            def fin_body(jj, carry):
                cl, sl = carry
                for u in range(_US):
                    j = jj * _US + u
                    dvec = d_v[pl.ds(j * _L, _L)]
                    less = dvec < t
                    cl = cl + jnp.where(less, 1.0, 0.0)
                    sl = sl + jnp.where(less, dvec, 0.0)
                return cl, sl
            cl, sl = lax.fori_loop(
                0, M // _US, fin_body,
                (jnp.zeros((_L,), jnp.float32), jnp.zeros((_L,), jnp.float32)))

            z = (sl + (wb - cl) * t) * (1.0 / wb)
            zb = lax.bitcast_convert_type(z, jnp.int32)
            y = lax.bitcast_convert_type(
                0x5F3759DF - lax.shift_right_logical(zb, 1), jnp.float32)
            for _r in range(3):
                y = y * (1.5 - 0.5 * z * y * y)
            o_v[...] = z * y
            pltpu.sync_copy(o_v, out_h.at[b, pl.ds(cb * _L, _L)])
            return 0

        lax.fori_loop(0, chunks_per_w, chunk_body, 0)

    return sc_kernel(xs, ys, gx, gy)


def _sc_dtm_full(inputs, grid_pts):
    B, M, _d = inputs.shape
    N = grid_pts.shape[0]
    n_chunks_pb = pl.cdiv(pl.cdiv(N, _L) * B, _NW) * _NW // B
    chunks_per_w = B * n_chunks_pb // _NW
    NP = n_chunks_pb * _L
    xs = inputs[:, :, 0]
    ys = inputs[:, :, 1]
    gx = jnp.zeros((NP,), jnp.float32).at[:N].set(grid_pts[:, 0])
    gy = jnp.zeros((NP,), jnp.float32).at[:N].set(grid_pts[:, 1])
    out = _sc_dtm(xs, ys, gx, gy,
                  n_chunks_pb=n_chunks_pb, chunks_per_w=chunks_per_w)
    return out[:, :N]


_SC_COLS = 560


def kernel(inputs, grid):
    out_sc = _sc_dtm_full(inputs, grid[:_SC_COLS])
    out_tc = _tc_dtm(inputs, grid[_SC_COLS:])
    return jnp.concatenate([out_sc, out_tc], axis=1)
